# baseline (speedup 1.0000x reference)
import functools
import jax
import jax.numpy as jnp
from jax import lax
from jax.experimental import pallas as pl
from jax.experimental.pallas import tpu as pltpu, tpu_sc as plsc

N = 10000
NP = 10240
E = 320000
EP = 322560
D = 128
G = 128
C = 10

NC = 2
NS = 16
NW = NC * NS
EPW = EP // NW
CH = 80
NCHUNK = EPW // CH
RPT = NP // NS

_mesh = plsc.VectorSubcoreMesh(
    core_axis_name="c", subcore_axis_name="s", num_cores=NC, num_subcores=NS)


NBUF = 3
AB = 2


def _make_agg(width):
    @functools.partial(
        pl.kernel,
        out_type=jax.ShapeDtypeStruct((NC * NP, width), jnp.float32),
        mesh=_mesh,
        scratch_types=[
            pltpu.VMEM((EPW,), jnp.int32),
            pltpu.VMEM((AB, CH), jnp.int32),
            pltpu.VMEM((AB * CH, width), jnp.float32),
            pltpu.VMEM_SHARED((NP, width), jnp.float32),
        ] + [pltpu.SemaphoreType.DMA] * AB,
    )
    def agg(hs_hbm, src_hbm, dst_hbm, out_hbm, srcall, dstall, rows, acc, *gsem):
        c = lax.axis_index("c")
        s = lax.axis_index("s")
        w = c * NS + s
        ebase = w * EPW
        pltpu.sync_copy(hs_hbm.at[pl.ds(s * RPT, RPT)], acc.at[pl.ds(s * RPT, RPT)])
        pltpu.sync_copy(src_hbm.at[pl.ds(ebase, EPW)], srcall)

        def fire(g, b):
            pltpu.async_copy(dst_hbm.at[pl.ds(ebase + g * CH, CH)],
                             dstall.at[b], gsem[b])
            pltpu.async_copy(hs_hbm.at[srcall.at[pl.ds(g * CH, CH)]],
                             rows.at[pl.ds(b * CH, CH)], gsem[b])

        def wait(b):
            pltpu.make_async_copy(dst_hbm.at[pl.ds(ebase, CH)],
                                  dstall.at[b], gsem[b]).wait()
            pltpu.make_async_copy(hs_hbm.at[pl.ds(0, CH)],
                                  rows.at[pl.ds(b * CH, CH)], gsem[b]).wait()

        def scat(b):
            pltpu.sync_copy(rows.at[pl.ds(b * CH, CH)],
                            acc.at[dstall.at[b]], add=True)

        for b in range(AB):
            fire(b, b)
        plsc.subcore_barrier()

        def step(j, carry):
            for b in range(AB):
                wait(b)
                scat(b)
                fire(j * AB + b + AB, b)
            return carry

        lax.fori_loop(0, NCHUNK // AB - 1, step, 0)
        for b in range(AB):
            wait(b)
            scat(b)
        plsc.subcore_barrier()
        pltpu.sync_copy(acc.at[pl.ds(s * RPT, RPT)],
                        out_hbm.at[pl.ds(c * NP + s * RPT, RPT)])

    return agg


_agg = _make_agg(D)


@functools.partial(
    pl.kernel,
    out_type=jax.ShapeDtypeStruct((NC * NP, D), jnp.float32),
    mesh=_mesh,
    scratch_types=[
        pltpu.VMEM((NBUF, CH), jnp.int32),
        pltpu.VMEM((CH, D), jnp.float32),
        pltpu.VMEM_SHARED((NP, D), jnp.float32),
    ] + [pltpu.SemaphoreType.DMA] * NBUF,
)
def _deg(dst_hbm, out_hbm, dstall, onesv, acc, *dsem):
    c = lax.axis_index("c")
    s = lax.axis_index("s")
    w = c * NS + s
    ebase = w * EPW
    zeros = jnp.zeros((16,), jnp.float32)
    ones = jnp.ones((16,), jnp.float32)

    def zb(j, carry):
        onesv[j // 8, pl.ds((j % 8) * 16, 16)] = zeros
        return carry
    lax.fori_loop(0, CH * 8, zb, 0)

    def zc(k, carry):
        pltpu.sync_copy(onesv, acc.at[pl.ds(s * RPT + k * CH, CH)])
        return carry
    lax.fori_loop(0, RPT // CH, zc, 0)

    def ob(j, carry):
        onesv[j // 8, pl.ds((j % 8) * 16, 16)] = ones
        return carry
    lax.fori_loop(0, CH * 8, ob, 0)

    def fire(g, b):
        pltpu.async_copy(dst_hbm.at[pl.ds(ebase + g * CH, CH)],
                         dstall.at[b], dsem[b])

    def wait(b):
        pltpu.make_async_copy(dst_hbm.at[pl.ds(ebase, CH)],
                              dstall.at[b], dsem[b]).wait()

    def scat(b):
        pltpu.sync_copy(onesv, acc.at[dstall.at[b]], add=True)

    for b in range(NBUF):
        fire(b, b)
    plsc.subcore_barrier()

    def step(j, carry):
        for b in range(NBUF):
            wait(b)
            scat(b)
            fire(j * NBUF + b + NBUF, b)
        return carry

    lax.fori_loop(0, NCHUNK // NBUF - 1, step, 0)
    for b in range(NBUF):
        wait(b)
        scat(b)
    plsc.subcore_barrier()
    pltpu.sync_copy(acc.at[pl.ds(s * RPT, RPT)],
                    out_hbm.at[pl.ds(c * NP + s * RPT, RPT)])


BN = 1024
NG = NP // BN
_PREC = lax.Precision.HIGHEST


def _tch_body(x_ref, w_ref, o_ref):
    o_ref[...] = jnp.dot(x_ref[...], w_ref[...], precision=_PREC,
                         preferred_element_type=jnp.float32)


_tch = pl.pallas_call(
    _tch_body,
    grid=(NG,),
    in_specs=[
        pl.BlockSpec((BN, D), lambda i: (i, 0)),
        pl.BlockSpec((D, D), lambda i: (0, 0)),
    ],
    out_specs=pl.BlockSpec((BN, D), lambda i: (i, 0)),
    out_shape=jax.ShapeDtypeStruct((NP, D), jnp.float32),
)


def _tcs_body(h_ref, d0_ref, d1_ref, o_ref):
    dinv = lax.rsqrt(d0_ref[...] + d1_ref[...] + 1.0)
    o_ref[...] = h_ref[...] * dinv


_tcs = pl.pallas_call(
    _tcs_body,
    grid=(NG,),
    in_specs=[
        pl.BlockSpec((BN, D), lambda i: (i, 0)),
        pl.BlockSpec((BN, 1), lambda i: (i, 0)),
        pl.BlockSpec((BN, 1), lambda i: (i + NG, 0)),
    ],
    out_specs=pl.BlockSpec((BN, D), lambda i: (i, 0)),
    out_shape=jax.ShapeDtypeStruct((NP, D), jnp.float32),
)


def _tc1_body(x_ref, w_ref, d0_ref, d1_ref, o_ref):
    dinv = lax.rsqrt(d0_ref[...] + d1_ref[...] + 1.0)
    h = jnp.dot(x_ref[...], w_ref[...], precision=_PREC,
                preferred_element_type=jnp.float32)
    o_ref[...] = h * dinv


_tc1 = pl.pallas_call(
    _tc1_body,
    grid=(NG,),
    in_specs=[
        pl.BlockSpec((BN, D), lambda i: (i, 0)),
        pl.BlockSpec((D, D), lambda i: (0, 0)),
        pl.BlockSpec((BN, 1), lambda i: (i, 0)),
        pl.BlockSpec((BN, 1), lambda i: (i + NG, 0)),
    ],
    out_specs=pl.BlockSpec((BN, D), lambda i: (i, 0)),
    out_shape=jax.ShapeDtypeStruct((NP, D), jnp.float32),
)


def _tcmid_body(p0_ref, p1_ref, hs_ref, d0_ref, d1_ref, b_ref, w_ref, o_ref):
    dinv = lax.rsqrt(d0_ref[...] + d1_ref[...] + 1.0)
    agg = (p0_ref[...] + p1_ref[...] - hs_ref[...]) * dinv + b_ref[...]
    a = jnp.maximum(agg, 0.0)
    o_ref[...] = jnp.dot(a, w_ref[...], precision=_PREC,
                         preferred_element_type=jnp.float32) * dinv


_tcmid = pl.pallas_call(
    _tcmid_body,
    grid=(NG,),
    in_specs=[
        pl.BlockSpec((BN, D), lambda i: (i, 0)),
        pl.BlockSpec((BN, D), lambda i: (i + NG, 0)),
        pl.BlockSpec((BN, D), lambda i: (i, 0)),
        pl.BlockSpec((BN, 1), lambda i: (i, 0)),
        pl.BlockSpec((BN, 1), lambda i: (i + NG, 0)),
        pl.BlockSpec((1, D), lambda i: (0, 0)),
        pl.BlockSpec((D, D), lambda i: (0, 0)),
    ],
    out_specs=pl.BlockSpec((BN, D), lambda i: (i, 0)),
    out_shape=jax.ShapeDtypeStruct((NP, D), jnp.float32),
)


def _tcf_body(p0_ref, p1_ref, hs_ref, d0_ref, d1_ref, b_ref, batch_ref,
              wl_ref, bl_ref, o_ref, sums, counts):
    i = pl.program_id(0)

    @pl.when(i == 0)
    def _():
        sums[...] = jnp.zeros_like(sums)
        counts[...] = jnp.zeros_like(counts)

    dinv = lax.rsqrt(d0_ref[...] + d1_ref[...] + 1.0)
    h = (p0_ref[...] + p1_ref[...] - hs_ref[...]) * dinv + b_ref[...]
    m = (batch_ref[...] == lax.broadcasted_iota(jnp.int32, (BN, G), 1)
         ).astype(jnp.float32)
    sums[...] += lax.dot_general(m, h, (((0,), (0,)), ((), ())),
                                 precision=_PREC,
                                 preferred_element_type=jnp.float32)
    counts[...] += lax.dot_general(m, jnp.ones((BN, 1), jnp.float32),
                                   (((0,), (0,)), ((), ())),
                                   precision=_PREC,
                                   preferred_element_type=jnp.float32)

    @pl.when(i == NG - 1)
    def _():
        pooled = sums[...] / jnp.maximum(counts[...], 1.0)
        o_ref[...] = jnp.dot(pooled, wl_ref[...], precision=_PREC,
                             preferred_element_type=jnp.float32) + bl_ref[...]


_tcf = pl.pallas_call(
    _tcf_body,
    grid=(NG,),
    in_specs=[
        pl.BlockSpec((BN, D), lambda i: (i, 0)),
        pl.BlockSpec((BN, D), lambda i: (i + NG, 0)),
        pl.BlockSpec((BN, D), lambda i: (i, 0)),
        pl.BlockSpec((BN, 1), lambda i: (i, 0)),
        pl.BlockSpec((BN, 1), lambda i: (i + NG, 0)),
        pl.BlockSpec((1, D), lambda i: (0, 0)),
        pl.BlockSpec((BN, 1), lambda i: (i, 0)),
        pl.BlockSpec((D, C), lambda i: (0, 0)),
        pl.BlockSpec((1, C), lambda i: (0, 0)),
    ],
    out_specs=pl.BlockSpec((G, C), lambda i: (0, 0)),
    out_shape=jax.ShapeDtypeStruct((G, C), jnp.float32),
    scratch_shapes=[
        pltpu.VMEM((G, G), jnp.float32),
        pltpu.VMEM((G, 1), jnp.float32),
    ],
)


def kernel(x, edge_index, batch, W1, b1, W2, b2, W3, b3, Wlin, blin):
    epad = jnp.full((EP - E,), N, jnp.int32)
    src = jnp.concatenate([edge_index[0].astype(jnp.int32), epad])
    dst = jnp.concatenate([edge_index[1].astype(jnp.int32), epad])
    xp = jnp.pad(x, ((0, NP - N), (0, 0)))
    batchp = jnp.pad(batch.astype(jnp.int32), (0, NP - N), constant_values=G)

    degp = _deg(dst)
    batch2d = batchp.reshape(NP, 1)
    b1r = b1.reshape(1, D)
    b2r = b2.reshape(1, D)
    b3r = b3.reshape(1, D)
    blr = blin.reshape(1, C)

    dcol = degp[:, :1]
    hs1 = _tc1(xp, W1, dcol, dcol)
    p1 = _agg(hs1, src, dst)
    hs2 = _tcmid(p1, p1, hs1, dcol, dcol, b1r, W2)
    p2 = _agg(hs2, src, dst)
    hs3 = _tcmid(p2, p2, hs2, dcol, dcol, b2r, W3)
    p3 = _agg(hs3, src, dst)
    out = _tcf(p3, p3, hs3, dcol, dcol, b3r, batch2d, Wlin, blr)
    return out


# speedup vs baseline: 1.8909x; 1.8909x over previous
import functools
import jax
import jax.numpy as jnp
from jax import lax
from jax.experimental import pallas as pl
from jax.experimental.pallas import tpu as pltpu, tpu_sc as plsc

N = 10000
NP = 10240
E = 320000
EP = 322560
D = 128
G = 128
C = 10

NC = 2
NS = 16
NW = NC * NS
EPW = EP // NW
CH = 80
NCHUNK = EPW // CH
RPT = NP // NS

_mesh = plsc.VectorSubcoreMesh(
    core_axis_name="c", subcore_axis_name="s", num_cores=NC, num_subcores=NS)


NBUF = 3
AB = 3


def _make_agg(width):
    @functools.partial(
        pl.kernel,
        out_type=jax.ShapeDtypeStruct((NC * NP, width), jnp.float32),
        mesh=_mesh,
        scratch_types=[
            pltpu.VMEM((EPW,), jnp.int32),
            pltpu.VMEM((AB, CH), jnp.int32),
            pltpu.VMEM((AB * CH, width), jnp.float32),
            pltpu.VMEM_SHARED((NP, width), jnp.float32),
        ] + [pltpu.SemaphoreType.DMA] * AB,
    )
    def agg(hs_hbm, src_hbm, dst_hbm, out_hbm, srcall, dstall, rows, acc, *gsem):
        c = lax.axis_index("c")
        s = lax.axis_index("s")
        w = c * NS + s
        ebase = w * EPW
        pltpu.sync_copy(hs_hbm.at[pl.ds(s * RPT, RPT)], acc.at[pl.ds(s * RPT, RPT)])
        pltpu.sync_copy(src_hbm.at[pl.ds(ebase, EPW)], srcall)

        def fire(g, b):
            pltpu.async_copy(dst_hbm.at[pl.ds(ebase + g * CH, CH)],
                             dstall.at[b], gsem[b])
            pltpu.async_copy(hs_hbm.at[srcall.at[pl.ds(g * CH, CH)]],
                             rows.at[pl.ds(b * CH, CH)], gsem[b])

        def wait(b):
            pltpu.make_async_copy(dst_hbm.at[pl.ds(ebase, CH)],
                                  dstall.at[b], gsem[b]).wait()
            pltpu.make_async_copy(hs_hbm.at[pl.ds(0, CH)],
                                  rows.at[pl.ds(b * CH, CH)], gsem[b]).wait()

        def scat(b):
            pltpu.sync_copy(rows.at[pl.ds(b * CH, CH)],
                            acc.at[dstall.at[b]], add=True)

        for b in range(AB):
            fire(b, b)
        plsc.subcore_barrier()

        def step(j, carry):
            for b in range(AB):
                wait(b)
                scat(b)
                fire(j * AB + b + AB, b)
            return carry

        lax.fori_loop(0, NCHUNK // AB - 1, step, 0)
        for b in range(AB):
            wait(b)
            scat(b)
        plsc.subcore_barrier()
        pltpu.sync_copy(acc.at[pl.ds(s * RPT, RPT)],
                        out_hbm.at[pl.ds(c * NP + s * RPT, RPT)])

    return agg


_agg = _make_agg(D)


@functools.partial(
    pl.kernel,
    out_type=jax.ShapeDtypeStruct((NC * NP, D), jnp.float32),
    mesh=_mesh,
    scratch_types=[
        pltpu.VMEM((NBUF, CH), jnp.int32),
        pltpu.VMEM((CH, D), jnp.float32),
        pltpu.VMEM_SHARED((NP, D), jnp.float32),
    ] + [pltpu.SemaphoreType.DMA] * NBUF,
)
def _deg(dst_hbm, out_hbm, dstall, onesv, acc, *dsem):
    c = lax.axis_index("c")
    s = lax.axis_index("s")
    w = c * NS + s
    ebase = w * EPW
    zeros = jnp.zeros((16,), jnp.float32)
    ones = jnp.ones((16,), jnp.float32)

    def zb(j, carry):
        onesv[j // 8, pl.ds((j % 8) * 16, 16)] = zeros
        return carry
    lax.fori_loop(0, CH * 8, zb, 0)

    def zc(k, carry):
        pltpu.sync_copy(onesv, acc.at[pl.ds(s * RPT + k * CH, CH)])
        return carry
    lax.fori_loop(0, RPT // CH, zc, 0)

    def ob(j, carry):
        onesv[j // 8, pl.ds((j % 8) * 16, 16)] = ones
        return carry
    lax.fori_loop(0, CH * 8, ob, 0)

    def fire(g, b):
        pltpu.async_copy(dst_hbm.at[pl.ds(ebase + g * CH, CH)],
                         dstall.at[b], dsem[b])

    def wait(b):
        pltpu.make_async_copy(dst_hbm.at[pl.ds(ebase, CH)],
                              dstall.at[b], dsem[b]).wait()

    def scat(b):
        pltpu.sync_copy(onesv, acc.at[dstall.at[b]], add=True)

    for b in range(NBUF):
        fire(b, b)
    plsc.subcore_barrier()

    def step(j, carry):
        for b in range(NBUF):
            wait(b)
            scat(b)
            fire(j * NBUF + b + NBUF, b)
        return carry

    lax.fori_loop(0, NCHUNK // NBUF - 1, step, 0)
    for b in range(NBUF):
        wait(b)
        scat(b)
    plsc.subcore_barrier()
    pltpu.sync_copy(acc.at[pl.ds(s * RPT, RPT)],
                    out_hbm.at[pl.ds(c * NP + s * RPT, RPT)])


BN = 1024
NG = NP // BN
_PREC = lax.Precision.HIGHEST


def _tch_body(x_ref, w_ref, o_ref):
    o_ref[...] = jnp.dot(x_ref[...], w_ref[...], precision=_PREC,
                         preferred_element_type=jnp.float32)


_tch = pl.pallas_call(
    _tch_body,
    grid=(NG,),
    in_specs=[
        pl.BlockSpec((BN, D), lambda i: (i, 0)),
        pl.BlockSpec((D, D), lambda i: (0, 0)),
    ],
    out_specs=pl.BlockSpec((BN, D), lambda i: (i, 0)),
    out_shape=jax.ShapeDtypeStruct((NP, D), jnp.float32),
)


def _tcs_body(h_ref, d0_ref, d1_ref, o_ref):
    dinv = lax.rsqrt(d0_ref[...] + d1_ref[...] + 1.0)
    o_ref[...] = h_ref[...] * dinv


_tcs = pl.pallas_call(
    _tcs_body,
    grid=(NG,),
    in_specs=[
        pl.BlockSpec((BN, D), lambda i: (i, 0)),
        pl.BlockSpec((BN, 1), lambda i: (i, 0)),
        pl.BlockSpec((BN, 1), lambda i: (i + NG, 0)),
    ],
    out_specs=pl.BlockSpec((BN, D), lambda i: (i, 0)),
    out_shape=jax.ShapeDtypeStruct((NP, D), jnp.float32),
)


def _tc1_body(x_ref, w_ref, d0_ref, d1_ref, o_ref):
    dinv = lax.rsqrt(d0_ref[...] + d1_ref[...] + 1.0)
    h = jnp.dot(x_ref[...], w_ref[...], precision=_PREC,
                preferred_element_type=jnp.float32)
    o_ref[...] = h * dinv


_tc1 = pl.pallas_call(
    _tc1_body,
    grid=(NG,),
    in_specs=[
        pl.BlockSpec((BN, D), lambda i: (i, 0)),
        pl.BlockSpec((D, D), lambda i: (0, 0)),
        pl.BlockSpec((BN, 1), lambda i: (i, 0)),
        pl.BlockSpec((BN, 1), lambda i: (i + NG, 0)),
    ],
    out_specs=pl.BlockSpec((BN, D), lambda i: (i, 0)),
    out_shape=jax.ShapeDtypeStruct((NP, D), jnp.float32),
)


def _tcmid_body(p0_ref, p1_ref, hs_ref, d0_ref, d1_ref, b_ref, w_ref, o_ref):
    dinv = lax.rsqrt(d0_ref[...] + d1_ref[...] + 1.0)
    agg = (p0_ref[...] + p1_ref[...] - hs_ref[...]) * dinv + b_ref[...]
    a = jnp.maximum(agg, 0.0)
    o_ref[...] = jnp.dot(a, w_ref[...], precision=_PREC,
                         preferred_element_type=jnp.float32) * dinv


_tcmid = pl.pallas_call(
    _tcmid_body,
    grid=(NG,),
    in_specs=[
        pl.BlockSpec((BN, D), lambda i: (i, 0)),
        pl.BlockSpec((BN, D), lambda i: (i + NG, 0)),
        pl.BlockSpec((BN, D), lambda i: (i, 0)),
        pl.BlockSpec((BN, 1), lambda i: (i, 0)),
        pl.BlockSpec((BN, 1), lambda i: (i + NG, 0)),
        pl.BlockSpec((1, D), lambda i: (0, 0)),
        pl.BlockSpec((D, D), lambda i: (0, 0)),
    ],
    out_specs=pl.BlockSpec((BN, D), lambda i: (i, 0)),
    out_shape=jax.ShapeDtypeStruct((NP, D), jnp.float32),
)


def _tcf_body(p0_ref, p1_ref, hs_ref, d0_ref, d1_ref, b_ref, batch_ref,
              wl_ref, bl_ref, o_ref, sums, counts):
    i = pl.program_id(0)

    @pl.when(i == 0)
    def _():
        sums[...] = jnp.zeros_like(sums)
        counts[...] = jnp.zeros_like(counts)

    dinv = lax.rsqrt(d0_ref[...] + d1_ref[...] + 1.0)
    h = (p0_ref[...] + p1_ref[...] - hs_ref[...]) * dinv + b_ref[...]
    m = (batch_ref[...] == lax.broadcasted_iota(jnp.int32, (BN, G), 1)
         ).astype(jnp.float32)
    sums[...] += lax.dot_general(m, h, (((0,), (0,)), ((), ())),
                                 precision=_PREC,
                                 preferred_element_type=jnp.float32)
    counts[...] += lax.dot_general(m, jnp.ones((BN, 1), jnp.float32),
                                   (((0,), (0,)), ((), ())),
                                   precision=_PREC,
                                   preferred_element_type=jnp.float32)

    @pl.when(i == NG - 1)
    def _():
        pooled = sums[...] / jnp.maximum(counts[...], 1.0)
        o_ref[...] = jnp.dot(pooled, wl_ref[...], precision=_PREC,
                             preferred_element_type=jnp.float32) + bl_ref[...]


_tcf = pl.pallas_call(
    _tcf_body,
    grid=(NG,),
    in_specs=[
        pl.BlockSpec((BN, D), lambda i: (i, 0)),
        pl.BlockSpec((BN, D), lambda i: (i + NG, 0)),
        pl.BlockSpec((BN, D), lambda i: (i, 0)),
        pl.BlockSpec((BN, 1), lambda i: (i, 0)),
        pl.BlockSpec((BN, 1), lambda i: (i + NG, 0)),
        pl.BlockSpec((1, D), lambda i: (0, 0)),
        pl.BlockSpec((BN, 1), lambda i: (i, 0)),
        pl.BlockSpec((D, C), lambda i: (0, 0)),
        pl.BlockSpec((1, C), lambda i: (0, 0)),
    ],
    out_specs=pl.BlockSpec((G, C), lambda i: (0, 0)),
    out_shape=jax.ShapeDtypeStruct((G, C), jnp.float32),
    scratch_shapes=[
        pltpu.VMEM((G, G), jnp.float32),
        pltpu.VMEM((G, 1), jnp.float32),
    ],
)


def kernel(x, edge_index, batch, W1, b1, W2, b2, W3, b3, Wlin, blin):
    eppt = (EP - E) // NW
    padrow = N + (jnp.arange(NW * eppt, dtype=jnp.int32).reshape(NW, eppt)
                  % (NP - N))
    src = jnp.concatenate(
        [edge_index[0].astype(jnp.int32).reshape(NW, E // NW), padrow],
        axis=1).reshape(EP)
    dst = jnp.concatenate(
        [edge_index[1].astype(jnp.int32).reshape(NW, E // NW), padrow],
        axis=1).reshape(EP)
    xp = jnp.pad(x, ((0, NP - N), (0, 0)))
    batchp = jnp.pad(batch.astype(jnp.int32), (0, NP - N), constant_values=G)

    degp = _deg(dst)
    batch2d = batchp.reshape(NP, 1)
    b1r = b1.reshape(1, D)
    b2r = b2.reshape(1, D)
    b3r = b3.reshape(1, D)
    blr = blin.reshape(1, C)

    dcol = degp[:, :1]
    hs1 = _tc1(xp, W1, dcol, dcol)
    p1 = _agg(hs1, src, dst)
    hs2 = _tcmid(p1, p1, hs1, dcol, dcol, b1r, W2)
    p2 = _agg(hs2, src, dst)
    hs3 = _tcmid(p2, p2, hs2, dcol, dcol, b2r, W3)
    p3 = _agg(hs3, src, dst)
    out = _tcf(p3, p3, hs3, dcol, dcol, b3r, batch2d, Wlin, blr)
    return out


# speedup vs baseline: 1.9088x; 1.0095x over previous
import functools
import jax
import jax.numpy as jnp
from jax import lax
from jax.experimental import pallas as pl
from jax.experimental.pallas import tpu as pltpu, tpu_sc as plsc

N = 10000
NP = 10240
E = 320000
EP = 322560
D = 128
G = 128
C = 10

NC = 2
NS = 16
NW = NC * NS
EPW = EP // NW
CH = 96
NCHUNK = EPW // CH
RPT = NP // NS

_mesh = plsc.VectorSubcoreMesh(
    core_axis_name="c", subcore_axis_name="s", num_cores=NC, num_subcores=NS)


NBUF = 3
AB = 3


def _make_agg(width):
    @functools.partial(
        pl.kernel,
        out_type=jax.ShapeDtypeStruct((NC * NP, width), jnp.float32),
        mesh=_mesh,
        scratch_types=[
            pltpu.VMEM((EPW,), jnp.int32),
            pltpu.VMEM((AB, CH), jnp.int32),
            pltpu.VMEM((AB * CH, width), jnp.float32),
            pltpu.VMEM_SHARED((NP, width), jnp.float32),
        ] + [pltpu.SemaphoreType.DMA] * AB,
    )
    def agg(hs_hbm, src_hbm, dst_hbm, out_hbm, srcall, dstall, rows, acc, *gsem):
        c = lax.axis_index("c")
        s = lax.axis_index("s")
        w = c * NS + s
        ebase = w * EPW
        pltpu.sync_copy(hs_hbm.at[pl.ds(s * RPT, RPT)], acc.at[pl.ds(s * RPT, RPT)])
        pltpu.sync_copy(src_hbm.at[pl.ds(ebase, EPW)], srcall)

        def fire(g, b):
            pltpu.async_copy(dst_hbm.at[pl.ds(ebase + g * CH, CH)],
                             dstall.at[b], gsem[b])
            pltpu.async_copy(hs_hbm.at[srcall.at[pl.ds(g * CH, CH)]],
                             rows.at[pl.ds(b * CH, CH)], gsem[b])

        def wait(b):
            pltpu.make_async_copy(dst_hbm.at[pl.ds(ebase, CH)],
                                  dstall.at[b], gsem[b]).wait()
            pltpu.make_async_copy(hs_hbm.at[pl.ds(0, CH)],
                                  rows.at[pl.ds(b * CH, CH)], gsem[b]).wait()

        def scat(b):
            pltpu.sync_copy(rows.at[pl.ds(b * CH, CH)],
                            acc.at[dstall.at[b]], add=True)

        for b in range(AB):
            fire(b, b)
        plsc.subcore_barrier()

        def step(j, carry):
            for b in range(AB):
                wait(b)
                scat(b)
                fire(j * AB + b + AB, b)
            return carry

        lax.fori_loop(0, NCHUNK // AB - 1, step, 0)
        for b in range(AB):
            wait(b)
            scat(b)
        plsc.subcore_barrier()
        pltpu.sync_copy(acc.at[pl.ds(s * RPT, RPT)],
                        out_hbm.at[pl.ds(c * NP + s * RPT, RPT)])

    return agg


_agg = _make_agg(D)


@functools.partial(
    pl.kernel,
    out_type=jax.ShapeDtypeStruct((NC * NP, D), jnp.float32),
    mesh=_mesh,
    scratch_types=[
        pltpu.VMEM((NBUF, CH), jnp.int32),
        pltpu.VMEM((CH, D), jnp.float32),
        pltpu.VMEM_SHARED((NP, D), jnp.float32),
    ] + [pltpu.SemaphoreType.DMA] * NBUF,
)
def _deg(dst_hbm, out_hbm, dstall, onesv, acc, *dsem):
    c = lax.axis_index("c")
    s = lax.axis_index("s")
    w = c * NS + s
    ebase = w * EPW
    zeros = jnp.zeros((16,), jnp.float32)
    ones = jnp.ones((16,), jnp.float32)

    def zb(j, carry):
        onesv[j // 8, pl.ds((j % 8) * 16, 16)] = zeros
        return carry
    lax.fori_loop(0, CH * 8, zb, 0)

    def zc(k, carry):
        pltpu.sync_copy(onesv, acc.at[pl.ds(s * RPT + k * CH, CH)])
        return carry
    lax.fori_loop(0, RPT // CH, zc, 0)

    def ob(j, carry):
        onesv[j // 8, pl.ds((j % 8) * 16, 16)] = ones
        return carry
    lax.fori_loop(0, CH * 8, ob, 0)

    def fire(g, b):
        pltpu.async_copy(dst_hbm.at[pl.ds(ebase + g * CH, CH)],
                         dstall.at[b], dsem[b])

    def wait(b):
        pltpu.make_async_copy(dst_hbm.at[pl.ds(ebase, CH)],
                              dstall.at[b], dsem[b]).wait()

    def scat(b):
        pltpu.sync_copy(onesv, acc.at[dstall.at[b]], add=True)

    for b in range(NBUF):
        fire(b, b)
    plsc.subcore_barrier()

    def step(j, carry):
        for b in range(NBUF):
            wait(b)
            scat(b)
            fire(j * NBUF + b + NBUF, b)
        return carry

    lax.fori_loop(0, NCHUNK // NBUF - 1, step, 0)
    for b in range(NBUF):
        wait(b)
        scat(b)
    plsc.subcore_barrier()
    pltpu.sync_copy(acc.at[pl.ds(s * RPT, RPT)],
                    out_hbm.at[pl.ds(c * NP + s * RPT, RPT)])


BN = 1024
NG = NP // BN
_PREC = lax.Precision.HIGHEST


def _tch_body(x_ref, w_ref, o_ref):
    o_ref[...] = jnp.dot(x_ref[...], w_ref[...], precision=_PREC,
                         preferred_element_type=jnp.float32)


_tch = pl.pallas_call(
    _tch_body,
    grid=(NG,),
    in_specs=[
        pl.BlockSpec((BN, D), lambda i: (i, 0)),
        pl.BlockSpec((D, D), lambda i: (0, 0)),
    ],
    out_specs=pl.BlockSpec((BN, D), lambda i: (i, 0)),
    out_shape=jax.ShapeDtypeStruct((NP, D), jnp.float32),
)


def _tcs_body(h_ref, d0_ref, d1_ref, o_ref):
    dinv = lax.rsqrt(d0_ref[...] + d1_ref[...] + 1.0)
    o_ref[...] = h_ref[...] * dinv


_tcs = pl.pallas_call(
    _tcs_body,
    grid=(NG,),
    in_specs=[
        pl.BlockSpec((BN, D), lambda i: (i, 0)),
        pl.BlockSpec((BN, 1), lambda i: (i, 0)),
        pl.BlockSpec((BN, 1), lambda i: (i + NG, 0)),
    ],
    out_specs=pl.BlockSpec((BN, D), lambda i: (i, 0)),
    out_shape=jax.ShapeDtypeStruct((NP, D), jnp.float32),
)


def _tc1_body(x_ref, w_ref, d0_ref, d1_ref, o_ref):
    dinv = lax.rsqrt(d0_ref[...] + d1_ref[...] + 1.0)
    h = jnp.dot(x_ref[...], w_ref[...], precision=_PREC,
                preferred_element_type=jnp.float32)
    o_ref[...] = h * dinv


_tc1 = pl.pallas_call(
    _tc1_body,
    grid=(NG,),
    in_specs=[
        pl.BlockSpec((BN, D), lambda i: (i, 0)),
        pl.BlockSpec((D, D), lambda i: (0, 0)),
        pl.BlockSpec((BN, 1), lambda i: (i, 0)),
        pl.BlockSpec((BN, 1), lambda i: (i + NG, 0)),
    ],
    out_specs=pl.BlockSpec((BN, D), lambda i: (i, 0)),
    out_shape=jax.ShapeDtypeStruct((NP, D), jnp.float32),
)


def _tcmid_body(p0_ref, p1_ref, hs_ref, d0_ref, d1_ref, b_ref, w_ref, o_ref):
    dinv = lax.rsqrt(d0_ref[...] + d1_ref[...] + 1.0)
    agg = (p0_ref[...] + p1_ref[...] - hs_ref[...]) * dinv + b_ref[...]
    a = jnp.maximum(agg, 0.0)
    o_ref[...] = jnp.dot(a, w_ref[...], precision=_PREC,
                         preferred_element_type=jnp.float32) * dinv


_tcmid = pl.pallas_call(
    _tcmid_body,
    grid=(NG,),
    in_specs=[
        pl.BlockSpec((BN, D), lambda i: (i, 0)),
        pl.BlockSpec((BN, D), lambda i: (i + NG, 0)),
        pl.BlockSpec((BN, D), lambda i: (i, 0)),
        pl.BlockSpec((BN, 1), lambda i: (i, 0)),
        pl.BlockSpec((BN, 1), lambda i: (i + NG, 0)),
        pl.BlockSpec((1, D), lambda i: (0, 0)),
        pl.BlockSpec((D, D), lambda i: (0, 0)),
    ],
    out_specs=pl.BlockSpec((BN, D), lambda i: (i, 0)),
    out_shape=jax.ShapeDtypeStruct((NP, D), jnp.float32),
)


def _tcf_body(p0_ref, p1_ref, hs_ref, d0_ref, d1_ref, b_ref, batch_ref,
              wl_ref, bl_ref, o_ref, sums, counts):
    i = pl.program_id(0)

    @pl.when(i == 0)
    def _():
        sums[...] = jnp.zeros_like(sums)
        counts[...] = jnp.zeros_like(counts)

    dinv = lax.rsqrt(d0_ref[...] + d1_ref[...] + 1.0)
    h = (p0_ref[...] + p1_ref[...] - hs_ref[...]) * dinv + b_ref[...]
    m = (batch_ref[...] == lax.broadcasted_iota(jnp.int32, (BN, G), 1)
         ).astype(jnp.float32)
    sums[...] += lax.dot_general(m, h, (((0,), (0,)), ((), ())),
                                 precision=_PREC,
                                 preferred_element_type=jnp.float32)
    counts[...] += lax.dot_general(m, jnp.ones((BN, 1), jnp.float32),
                                   (((0,), (0,)), ((), ())),
                                   precision=_PREC,
                                   preferred_element_type=jnp.float32)

    @pl.when(i == NG - 1)
    def _():
        pooled = sums[...] / jnp.maximum(counts[...], 1.0)
        o_ref[...] = jnp.dot(pooled, wl_ref[...], precision=_PREC,
                             preferred_element_type=jnp.float32) + bl_ref[...]


_tcf = pl.pallas_call(
    _tcf_body,
    grid=(NG,),
    in_specs=[
        pl.BlockSpec((BN, D), lambda i: (i, 0)),
        pl.BlockSpec((BN, D), lambda i: (i + NG, 0)),
        pl.BlockSpec((BN, D), lambda i: (i, 0)),
        pl.BlockSpec((BN, 1), lambda i: (i, 0)),
        pl.BlockSpec((BN, 1), lambda i: (i + NG, 0)),
        pl.BlockSpec((1, D), lambda i: (0, 0)),
        pl.BlockSpec((BN, 1), lambda i: (i, 0)),
        pl.BlockSpec((D, C), lambda i: (0, 0)),
        pl.BlockSpec((1, C), lambda i: (0, 0)),
    ],
    out_specs=pl.BlockSpec((G, C), lambda i: (0, 0)),
    out_shape=jax.ShapeDtypeStruct((G, C), jnp.float32),
    scratch_shapes=[
        pltpu.VMEM((G, G), jnp.float32),
        pltpu.VMEM((G, 1), jnp.float32),
    ],
)


def kernel(x, edge_index, batch, W1, b1, W2, b2, W3, b3, Wlin, blin):
    eppt = (EP - E) // NW
    padrow = N + (jnp.arange(NW * eppt, dtype=jnp.int32).reshape(NW, eppt)
                  % (NP - N))
    src = jnp.concatenate(
        [edge_index[0].astype(jnp.int32).reshape(NW, E // NW), padrow],
        axis=1).reshape(EP)
    dst = jnp.concatenate(
        [edge_index[1].astype(jnp.int32).reshape(NW, E // NW), padrow],
        axis=1).reshape(EP)
    xp = jnp.pad(x, ((0, NP - N), (0, 0)))
    batchp = jnp.pad(batch.astype(jnp.int32), (0, NP - N), constant_values=G)

    degp = _deg(dst)
    batch2d = batchp.reshape(NP, 1)
    b1r = b1.reshape(1, D)
    b2r = b2.reshape(1, D)
    b3r = b3.reshape(1, D)
    blr = blin.reshape(1, C)

    dcol = degp[:, :1]
    hs1 = _tc1(xp, W1, dcol, dcol)
    p1 = _agg(hs1, src, dst)
    hs2 = _tcmid(p1, p1, hs1, dcol, dcol, b1r, W2)
    p2 = _agg(hs2, src, dst)
    hs3 = _tcmid(p2, p2, hs2, dcol, dcol, b2r, W3)
    p3 = _agg(hs3, src, dst)
    out = _tcf(p3, p3, hs3, dcol, dcol, b3r, batch2d, Wlin, blr)
    return out


# speedup vs baseline: 1.9276x; 1.0098x over previous
import functools
import jax
import jax.numpy as jnp
from jax import lax
from jax.experimental import pallas as pl
from jax.experimental.pallas import tpu as pltpu, tpu_sc as plsc

N = 10000
NP = 10240
E = 320000
EP = 322560
D = 128
G = 128
C = 10

NC = 2
NS = 16
NW = NC * NS
EPW = EP // NW
CH = 96
NCHUNK = EPW // CH
RPT = NP // NS

_mesh = plsc.VectorSubcoreMesh(
    core_axis_name="c", subcore_axis_name="s", num_cores=NC, num_subcores=NS)


NBUF = 3
AB = 3


def _make_agg(width):
    @functools.partial(
        pl.kernel,
        out_type=jax.ShapeDtypeStruct((NC * NP, width), jnp.float32),
        mesh=_mesh,
        scratch_types=[
            pltpu.VMEM((EPW,), jnp.int32),
            pltpu.VMEM((AB, CH), jnp.int32),
            pltpu.VMEM((AB * CH, width), jnp.float32),
            pltpu.VMEM_SHARED((NP, width), jnp.float32),
        ] + [pltpu.SemaphoreType.DMA] * AB,
    )
    def agg(hs_hbm, src_hbm, dst_hbm, out_hbm, srcall, dstall, rows, acc, *gsem):
        c = lax.axis_index("c")
        s = lax.axis_index("s")
        w = c * NS + s
        ebase = w * EPW

        zeros = jnp.zeros((16,), jnp.float32)

        def zb(j, carry):
            rows[j // 8, pl.ds((j % 8) * 16, 16)] = zeros
            return carry
        lax.fori_loop(0, CH * 8, zb, 0)

        def zc(k, carry):
            pltpu.sync_copy(rows.at[pl.ds(0, CH)],
                            acc.at[pl.ds(s * RPT + k * CH, CH)])
            return carry
        lax.fori_loop(0, RPT // CH, zc, 0)
        if RPT % CH:
            pltpu.sync_copy(rows.at[pl.ds(0, RPT % CH)],
                            acc.at[pl.ds(s * RPT + (RPT // CH) * CH, RPT % CH)])
        pltpu.sync_copy(src_hbm.at[pl.ds(ebase, EPW)], srcall)

        def fire(g, b):
            pltpu.async_copy(dst_hbm.at[pl.ds(ebase + g * CH, CH)],
                             dstall.at[b], gsem[b])
            pltpu.async_copy(hs_hbm.at[srcall.at[pl.ds(g * CH, CH)]],
                             rows.at[pl.ds(b * CH, CH)], gsem[b])

        def wait(b):
            pltpu.make_async_copy(dst_hbm.at[pl.ds(ebase, CH)],
                                  dstall.at[b], gsem[b]).wait()
            pltpu.make_async_copy(hs_hbm.at[pl.ds(0, CH)],
                                  rows.at[pl.ds(b * CH, CH)], gsem[b]).wait()

        def scat(b):
            pltpu.sync_copy(rows.at[pl.ds(b * CH, CH)],
                            acc.at[dstall.at[b]], add=True)

        for b in range(AB):
            fire(b, b)
        plsc.subcore_barrier()

        def step(j, carry):
            for b in range(AB):
                wait(b)
                scat(b)
                fire(j * AB + b + AB, b)
            return carry

        lax.fori_loop(0, NCHUNK // AB - 1, step, 0)
        for b in range(AB):
            wait(b)
            scat(b)
        plsc.subcore_barrier()
        pltpu.sync_copy(acc.at[pl.ds(s * RPT, RPT)],
                        out_hbm.at[pl.ds(c * NP + s * RPT, RPT)])

    return agg


_agg = _make_agg(D)


@functools.partial(
    pl.kernel,
    out_type=jax.ShapeDtypeStruct((NC * NP, D), jnp.float32),
    mesh=_mesh,
    scratch_types=[
        pltpu.VMEM((NBUF, CH), jnp.int32),
        pltpu.VMEM((CH, D), jnp.float32),
        pltpu.VMEM_SHARED((NP, D), jnp.float32),
    ] + [pltpu.SemaphoreType.DMA] * NBUF,
)
def _deg(dst_hbm, out_hbm, dstall, onesv, acc, *dsem):
    c = lax.axis_index("c")
    s = lax.axis_index("s")
    w = c * NS + s
    ebase = w * EPW
    zeros = jnp.zeros((16,), jnp.float32)
    ones = jnp.ones((16,), jnp.float32)

    def zb(j, carry):
        onesv[j // 8, pl.ds((j % 8) * 16, 16)] = zeros
        return carry
    lax.fori_loop(0, CH * 8, zb, 0)

    def zc(k, carry):
        pltpu.sync_copy(onesv, acc.at[pl.ds(s * RPT + k * CH, CH)])
        return carry
    lax.fori_loop(0, RPT // CH, zc, 0)

    def ob(j, carry):
        onesv[j // 8, pl.ds((j % 8) * 16, 16)] = ones
        return carry
    lax.fori_loop(0, CH * 8, ob, 0)

    def fire(g, b):
        pltpu.async_copy(dst_hbm.at[pl.ds(ebase + g * CH, CH)],
                         dstall.at[b], dsem[b])

    def wait(b):
        pltpu.make_async_copy(dst_hbm.at[pl.ds(ebase, CH)],
                              dstall.at[b], dsem[b]).wait()

    def scat(b):
        pltpu.sync_copy(onesv, acc.at[dstall.at[b]], add=True)

    for b in range(NBUF):
        fire(b, b)
    plsc.subcore_barrier()

    def step(j, carry):
        for b in range(NBUF):
            wait(b)
            scat(b)
            fire(j * NBUF + b + NBUF, b)
        return carry

    lax.fori_loop(0, NCHUNK // NBUF - 1, step, 0)
    for b in range(NBUF):
        wait(b)
        scat(b)
    plsc.subcore_barrier()
    pltpu.sync_copy(acc.at[pl.ds(s * RPT, RPT)],
                    out_hbm.at[pl.ds(c * NP + s * RPT, RPT)])


BN = 1024
NG = NP // BN
_PREC = lax.Precision.HIGHEST


def _tch_body(x_ref, w_ref, o_ref):
    o_ref[...] = jnp.dot(x_ref[...], w_ref[...], precision=_PREC,
                         preferred_element_type=jnp.float32)


_tch = pl.pallas_call(
    _tch_body,
    grid=(NG,),
    in_specs=[
        pl.BlockSpec((BN, D), lambda i: (i, 0)),
        pl.BlockSpec((D, D), lambda i: (0, 0)),
    ],
    out_specs=pl.BlockSpec((BN, D), lambda i: (i, 0)),
    out_shape=jax.ShapeDtypeStruct((NP, D), jnp.float32),
)


def _tcs_body(h_ref, d0_ref, d1_ref, o_ref):
    dinv = lax.rsqrt(d0_ref[...] + d1_ref[...] + 1.0)
    o_ref[...] = h_ref[...] * dinv


_tcs = pl.pallas_call(
    _tcs_body,
    grid=(NG,),
    in_specs=[
        pl.BlockSpec((BN, D), lambda i: (i, 0)),
        pl.BlockSpec((BN, 1), lambda i: (i, 0)),
        pl.BlockSpec((BN, 1), lambda i: (i + NG, 0)),
    ],
    out_specs=pl.BlockSpec((BN, D), lambda i: (i, 0)),
    out_shape=jax.ShapeDtypeStruct((NP, D), jnp.float32),
)


def _tc1_body(x_ref, w_ref, d0_ref, d1_ref, o_ref):
    dinv = lax.rsqrt(d0_ref[...] + d1_ref[...] + 1.0)
    h = jnp.dot(x_ref[...], w_ref[...], precision=_PREC,
                preferred_element_type=jnp.float32)
    o_ref[...] = h * dinv


_tc1 = pl.pallas_call(
    _tc1_body,
    grid=(NG,),
    in_specs=[
        pl.BlockSpec((BN, D), lambda i: (i, 0)),
        pl.BlockSpec((D, D), lambda i: (0, 0)),
        pl.BlockSpec((BN, 1), lambda i: (i, 0)),
        pl.BlockSpec((BN, 1), lambda i: (i + NG, 0)),
    ],
    out_specs=pl.BlockSpec((BN, D), lambda i: (i, 0)),
    out_shape=jax.ShapeDtypeStruct((NP, D), jnp.float32),
)


def _tcmid_body(p0_ref, p1_ref, hs_ref, d0_ref, d1_ref, b_ref, w_ref, o_ref):
    dinv = lax.rsqrt(d0_ref[...] + d1_ref[...] + 1.0)
    agg = (p0_ref[...] + p1_ref[...] + hs_ref[...]) * dinv + b_ref[...]
    a = jnp.maximum(agg, 0.0)
    o_ref[...] = jnp.dot(a, w_ref[...], precision=_PREC,
                         preferred_element_type=jnp.float32) * dinv


_tcmid = pl.pallas_call(
    _tcmid_body,
    grid=(NG,),
    in_specs=[
        pl.BlockSpec((BN, D), lambda i: (i, 0)),
        pl.BlockSpec((BN, D), lambda i: (i + NG, 0)),
        pl.BlockSpec((BN, D), lambda i: (i, 0)),
        pl.BlockSpec((BN, 1), lambda i: (i, 0)),
        pl.BlockSpec((BN, 1), lambda i: (i + NG, 0)),
        pl.BlockSpec((1, D), lambda i: (0, 0)),
        pl.BlockSpec((D, D), lambda i: (0, 0)),
    ],
    out_specs=pl.BlockSpec((BN, D), lambda i: (i, 0)),
    out_shape=jax.ShapeDtypeStruct((NP, D), jnp.float32),
)


def _tcf_body(p0_ref, p1_ref, hs_ref, d0_ref, d1_ref, b_ref, batch_ref,
              wl_ref, bl_ref, o_ref, sums, counts):
    i = pl.program_id(0)

    @pl.when(i == 0)
    def _():
        sums[...] = jnp.zeros_like(sums)
        counts[...] = jnp.zeros_like(counts)

    dinv = lax.rsqrt(d0_ref[...] + d1_ref[...] + 1.0)
    h = (p0_ref[...] + p1_ref[...] + hs_ref[...]) * dinv + b_ref[...]
    m = (batch_ref[...] == lax.broadcasted_iota(jnp.int32, (BN, G), 1)
         ).astype(jnp.float32)
    sums[...] += lax.dot_general(m, h, (((0,), (0,)), ((), ())),
                                 precision=_PREC,
                                 preferred_element_type=jnp.float32)
    counts[...] += lax.dot_general(m, jnp.ones((BN, 1), jnp.float32),
                                   (((0,), (0,)), ((), ())),
                                   precision=_PREC,
                                   preferred_element_type=jnp.float32)

    @pl.when(i == NG - 1)
    def _():
        pooled = sums[...] / jnp.maximum(counts[...], 1.0)
        o_ref[...] = jnp.dot(pooled, wl_ref[...], precision=_PREC,
                             preferred_element_type=jnp.float32) + bl_ref[...]


_tcf = pl.pallas_call(
    _tcf_body,
    grid=(NG,),
    in_specs=[
        pl.BlockSpec((BN, D), lambda i: (i, 0)),
        pl.BlockSpec((BN, D), lambda i: (i + NG, 0)),
        pl.BlockSpec((BN, D), lambda i: (i, 0)),
        pl.BlockSpec((BN, 1), lambda i: (i, 0)),
        pl.BlockSpec((BN, 1), lambda i: (i + NG, 0)),
        pl.BlockSpec((1, D), lambda i: (0, 0)),
        pl.BlockSpec((BN, 1), lambda i: (i, 0)),
        pl.BlockSpec((D, C), lambda i: (0, 0)),
        pl.BlockSpec((1, C), lambda i: (0, 0)),
    ],
    out_specs=pl.BlockSpec((G, C), lambda i: (0, 0)),
    out_shape=jax.ShapeDtypeStruct((G, C), jnp.float32),
    scratch_shapes=[
        pltpu.VMEM((G, G), jnp.float32),
        pltpu.VMEM((G, 1), jnp.float32),
    ],
)


def kernel(x, edge_index, batch, W1, b1, W2, b2, W3, b3, Wlin, blin):
    eppt = (EP - E) // NW
    padrow = N + (jnp.arange(NW * eppt, dtype=jnp.int32).reshape(NW, eppt)
                  % (NP - N))
    src = jnp.concatenate(
        [edge_index[0].astype(jnp.int32).reshape(NW, E // NW), padrow],
        axis=1).reshape(EP)
    dst = jnp.concatenate(
        [edge_index[1].astype(jnp.int32).reshape(NW, E // NW), padrow],
        axis=1).reshape(EP)
    xp = jnp.pad(x, ((0, NP - N), (0, 0)))
    batchp = jnp.pad(batch.astype(jnp.int32), (0, NP - N), constant_values=G)

    degp = _deg(dst)
    batch2d = batchp.reshape(NP, 1)
    b1r = b1.reshape(1, D)
    b2r = b2.reshape(1, D)
    b3r = b3.reshape(1, D)
    blr = blin.reshape(1, C)

    dcol = degp[:, :1]
    h1 = _tch(xp, W1)
    hs1 = _tcs(h1, dcol, dcol)
    p1 = _agg(hs1, src, dst)
    hs2 = _tcmid(p1, p1, hs1, dcol, dcol, b1r, W2)
    p2 = _agg(hs2, src, dst)
    hs3 = _tcmid(p2, p2, hs2, dcol, dcol, b2r, W3)
    p3 = _agg(hs3, src, dst)
    out = _tcf(p3, p3, hs3, dcol, dcol, b3r, batch2d, Wlin, blr)
    return out


# speedup vs baseline: 1.9303x; 1.0014x over previous
import functools
import jax
import jax.numpy as jnp
from jax import lax
from jax.experimental import pallas as pl
from jax.experimental.pallas import tpu as pltpu, tpu_sc as plsc

N = 10000
NP = 10240
E = 320000
EP = 322560
D = 128
G = 128
C = 10

NC = 2
NS = 16
NW = NC * NS
EPW = EP // NW
CH = 96
NCHUNK = EPW // CH
RPT = NP // NS

_mesh = plsc.VectorSubcoreMesh(
    core_axis_name="c", subcore_axis_name="s", num_cores=NC, num_subcores=NS)


NBUF = 3
AB = 3


def _make_agg(width):
    @functools.partial(
        pl.kernel,
        out_type=jax.ShapeDtypeStruct((NC * NP, width), jnp.float32),
        mesh=_mesh,
        scratch_types=[
            pltpu.VMEM((EPW,), jnp.int32),
            pltpu.VMEM((AB, CH), jnp.int32),
            pltpu.VMEM((AB * CH, width), jnp.float32),
            pltpu.VMEM_SHARED((NP, width), jnp.float32),
        ] + [pltpu.SemaphoreType.DMA] * AB,
    )
    def agg(hs_hbm, src_hbm, dst_hbm, out_hbm, srcall, dstall, rows, acc, *gsem):
        c = lax.axis_index("c")
        s = lax.axis_index("s")
        w = c * NS + s
        ebase = w * EPW

        zeros = jnp.zeros((16,), jnp.float32)

        def zb(j, carry):
            rows[j // 8, pl.ds((j % 8) * 16, 16)] = zeros
            return carry
        lax.fori_loop(0, CH * 8, zb, 0)

        def zc(k, carry):
            pltpu.sync_copy(rows.at[pl.ds(0, CH)],
                            acc.at[pl.ds(s * RPT + k * CH, CH)])
            return carry
        lax.fori_loop(0, RPT // CH, zc, 0)
        if RPT % CH:
            pltpu.sync_copy(rows.at[pl.ds(0, RPT % CH)],
                            acc.at[pl.ds(s * RPT + (RPT // CH) * CH, RPT % CH)])
        pltpu.sync_copy(src_hbm.at[pl.ds(ebase, EPW)], srcall)

        def fire(g, b):
            pltpu.async_copy(dst_hbm.at[pl.ds(ebase + g * CH, CH)],
                             dstall.at[b], gsem[b])
            pltpu.async_copy(hs_hbm.at[srcall.at[pl.ds(g * CH, CH)]],
                             rows.at[pl.ds(b * CH, CH)], gsem[b])

        def wait(b):
            pltpu.make_async_copy(dst_hbm.at[pl.ds(ebase, CH)],
                                  dstall.at[b], gsem[b]).wait()
            pltpu.make_async_copy(hs_hbm.at[pl.ds(0, CH)],
                                  rows.at[pl.ds(b * CH, CH)], gsem[b]).wait()

        def scat(b):
            pltpu.sync_copy(rows.at[pl.ds(b * CH, CH)],
                            acc.at[dstall.at[b]], add=True)

        for b in range(AB):
            fire(b, b)
        plsc.subcore_barrier()

        def step(j, carry):
            for b in range(AB):
                wait(b)
                scat(b)
                fire(j * AB + b + AB, b)
            return carry

        lax.fori_loop(0, NCHUNK // AB - 1, step, 0)
        for b in range(AB):
            wait(b)
            scat(b)
        plsc.subcore_barrier()
        pltpu.sync_copy(acc.at[pl.ds(s * RPT, RPT)],
                        out_hbm.at[pl.ds(c * NP + s * RPT, RPT)])

    return agg


_agg = _make_agg(D)


@functools.partial(
    pl.kernel,
    out_type=jax.ShapeDtypeStruct((NC * NP, D), jnp.float32),
    mesh=_mesh,
    scratch_types=[
        pltpu.VMEM((NCHUNK, CH), jnp.int32),
        pltpu.VMEM((CH, D), jnp.float32),
        pltpu.VMEM_SHARED((NP, D), jnp.float32),
        pltpu.SemaphoreType.DMA,
    ],
)
def _deg(dst3_hbm, out_hbm, dstall, onesv, acc, ssem):
    c = lax.axis_index("c")
    s = lax.axis_index("s")
    w = c * NS + s
    zeros = jnp.zeros((16,), jnp.float32)
    ones = jnp.ones((16,), jnp.float32)

    def zb(j, carry):
        onesv[j // 8, pl.ds((j % 8) * 16, 16)] = zeros
        return carry
    lax.fori_loop(0, CH * 8, zb, 0)

    def zc(k, carry):
        pltpu.sync_copy(onesv, acc.at[pl.ds(s * RPT + k * CH, CH)])
        return carry
    lax.fori_loop(0, RPT // CH, zc, 0)
    if RPT % CH:
        pltpu.sync_copy(onesv.at[pl.ds(0, RPT % CH)],
                        acc.at[pl.ds(s * RPT + (RPT // CH) * CH, RPT % CH)])

    def ob(j, carry):
        onesv[j // 8, pl.ds((j % 8) * 16, 16)] = ones
        return carry
    lax.fori_loop(0, CH * 8, ob, 0)

    pltpu.sync_copy(dst3_hbm.at[w], dstall)
    plsc.subcore_barrier()

    def body(g, carry):
        pltpu.async_copy(onesv, acc.at[dstall.at[g]], ssem, add=True)
        return carry
    lax.fori_loop(0, NCHUNK, body, 0)

    def drain(g, carry):
        pltpu.make_async_copy(onesv, acc.at[dstall.at[0]], ssem).wait()
        return carry
    lax.fori_loop(0, NCHUNK, drain, 0)
    plsc.subcore_barrier()
    pltpu.sync_copy(acc.at[pl.ds(s * RPT, RPT)],
                    out_hbm.at[pl.ds(c * NP + s * RPT, RPT)])


BN = 1024
NG = NP // BN
_PREC = lax.Precision.HIGHEST


def _tch_body(x_ref, w_ref, o_ref):
    o_ref[...] = jnp.dot(x_ref[...], w_ref[...], precision=_PREC,
                         preferred_element_type=jnp.float32)


_tch = pl.pallas_call(
    _tch_body,
    grid=(NG,),
    in_specs=[
        pl.BlockSpec((BN, D), lambda i: (i, 0)),
        pl.BlockSpec((D, D), lambda i: (0, 0)),
    ],
    out_specs=pl.BlockSpec((BN, D), lambda i: (i, 0)),
    out_shape=jax.ShapeDtypeStruct((NP, D), jnp.float32),
)


def _tcs_body(h_ref, d0_ref, d1_ref, o_ref):
    dinv = lax.rsqrt(d0_ref[...] + d1_ref[...] + 1.0)
    o_ref[...] = h_ref[...] * dinv


_tcs = pl.pallas_call(
    _tcs_body,
    grid=(NG,),
    in_specs=[
        pl.BlockSpec((BN, D), lambda i: (i, 0)),
        pl.BlockSpec((BN, 1), lambda i: (i, 0)),
        pl.BlockSpec((BN, 1), lambda i: (i + NG, 0)),
    ],
    out_specs=pl.BlockSpec((BN, D), lambda i: (i, 0)),
    out_shape=jax.ShapeDtypeStruct((NP, D), jnp.float32),
)


def _tc1_body(x_ref, w_ref, d0_ref, d1_ref, o_ref):
    dinv = lax.rsqrt(d0_ref[...] + d1_ref[...] + 1.0)
    h = jnp.dot(x_ref[...], w_ref[...], precision=_PREC,
                preferred_element_type=jnp.float32)
    o_ref[...] = h * dinv


_tc1 = pl.pallas_call(
    _tc1_body,
    grid=(NG,),
    in_specs=[
        pl.BlockSpec((BN, D), lambda i: (i, 0)),
        pl.BlockSpec((D, D), lambda i: (0, 0)),
        pl.BlockSpec((BN, 1), lambda i: (i, 0)),
        pl.BlockSpec((BN, 1), lambda i: (i + NG, 0)),
    ],
    out_specs=pl.BlockSpec((BN, D), lambda i: (i, 0)),
    out_shape=jax.ShapeDtypeStruct((NP, D), jnp.float32),
)


def _tcmid_body(p0_ref, p1_ref, hs_ref, d0_ref, d1_ref, b_ref, w_ref, o_ref):
    dinv = lax.rsqrt(d0_ref[...] + d1_ref[...] + 1.0)
    agg = (p0_ref[...] + p1_ref[...] + hs_ref[...]) * dinv + b_ref[...]
    a = jnp.maximum(agg, 0.0)
    o_ref[...] = jnp.dot(a, w_ref[...], precision=_PREC,
                         preferred_element_type=jnp.float32) * dinv


_tcmid = pl.pallas_call(
    _tcmid_body,
    grid=(NG,),
    in_specs=[
        pl.BlockSpec((BN, D), lambda i: (i, 0)),
        pl.BlockSpec((BN, D), lambda i: (i + NG, 0)),
        pl.BlockSpec((BN, D), lambda i: (i, 0)),
        pl.BlockSpec((BN, 1), lambda i: (i, 0)),
        pl.BlockSpec((BN, 1), lambda i: (i + NG, 0)),
        pl.BlockSpec((1, D), lambda i: (0, 0)),
        pl.BlockSpec((D, D), lambda i: (0, 0)),
    ],
    out_specs=pl.BlockSpec((BN, D), lambda i: (i, 0)),
    out_shape=jax.ShapeDtypeStruct((NP, D), jnp.float32),
)


def _tcf_body(p0_ref, p1_ref, hs_ref, d0_ref, d1_ref, b_ref, batch_ref,
              wl_ref, bl_ref, o_ref, sums, counts):
    i = pl.program_id(0)

    @pl.when(i == 0)
    def _():
        sums[...] = jnp.zeros_like(sums)
        counts[...] = jnp.zeros_like(counts)

    dinv = lax.rsqrt(d0_ref[...] + d1_ref[...] + 1.0)
    h = (p0_ref[...] + p1_ref[...] + hs_ref[...]) * dinv + b_ref[...]
    m = (batch_ref[...] == lax.broadcasted_iota(jnp.int32, (BN, G), 1)
         ).astype(jnp.float32)
    sums[...] += lax.dot_general(m, h, (((0,), (0,)), ((), ())),
                                 precision=_PREC,
                                 preferred_element_type=jnp.float32)
    counts[...] += lax.dot_general(m, jnp.ones((BN, 1), jnp.float32),
                                   (((0,), (0,)), ((), ())),
                                   precision=_PREC,
                                   preferred_element_type=jnp.float32)

    @pl.when(i == NG - 1)
    def _():
        pooled = sums[...] / jnp.maximum(counts[...], 1.0)
        o_ref[...] = jnp.dot(pooled, wl_ref[...], precision=_PREC,
                             preferred_element_type=jnp.float32) + bl_ref[...]


_tcf = pl.pallas_call(
    _tcf_body,
    grid=(NG,),
    in_specs=[
        pl.BlockSpec((BN, D), lambda i: (i, 0)),
        pl.BlockSpec((BN, D), lambda i: (i + NG, 0)),
        pl.BlockSpec((BN, D), lambda i: (i, 0)),
        pl.BlockSpec((BN, 1), lambda i: (i, 0)),
        pl.BlockSpec((BN, 1), lambda i: (i + NG, 0)),
        pl.BlockSpec((1, D), lambda i: (0, 0)),
        pl.BlockSpec((BN, 1), lambda i: (i, 0)),
        pl.BlockSpec((D, C), lambda i: (0, 0)),
        pl.BlockSpec((1, C), lambda i: (0, 0)),
    ],
    out_specs=pl.BlockSpec((G, C), lambda i: (0, 0)),
    out_shape=jax.ShapeDtypeStruct((G, C), jnp.float32),
    scratch_shapes=[
        pltpu.VMEM((G, G), jnp.float32),
        pltpu.VMEM((G, 1), jnp.float32),
    ],
)


def kernel(x, edge_index, batch, W1, b1, W2, b2, W3, b3, Wlin, blin):
    eppt = (EP - E) // NW
    padrow = N + (jnp.arange(NW * eppt, dtype=jnp.int32).reshape(NW, eppt)
                  % (NP - N))
    src = jnp.concatenate(
        [edge_index[0].astype(jnp.int32).reshape(NW, E // NW), padrow],
        axis=1).reshape(EP)
    dst = jnp.concatenate(
        [edge_index[1].astype(jnp.int32).reshape(NW, E // NW), padrow],
        axis=1).reshape(EP)
    xp = jnp.pad(x, ((0, NP - N), (0, 0)))
    batchp = jnp.pad(batch.astype(jnp.int32), (0, NP - N), constant_values=G)

    degp = _deg(dst.reshape(NW, NCHUNK, CH))
    batch2d = batchp.reshape(NP, 1)
    b1r = b1.reshape(1, D)
    b2r = b2.reshape(1, D)
    b3r = b3.reshape(1, D)
    blr = blin.reshape(1, C)

    dcol = degp[:, :1]
    h1 = _tch(xp, W1)
    hs1 = _tcs(h1, dcol, dcol)
    p1 = _agg(hs1, src, dst)
    hs2 = _tcmid(p1, p1, hs1, dcol, dcol, b1r, W2)
    p2 = _agg(hs2, src, dst)
    hs3 = _tcmid(p2, p2, hs2, dcol, dcol, b2r, W3)
    p3 = _agg(hs3, src, dst)
    out = _tcf(p3, p3, hs3, dcol, dcol, b3r, batch2d, Wlin, blr)
    return out
